# baseline (device time: 43636 ns/iter reference)
import os

import jax
import jax.numpy as jnp
from jax import lax
from jax.experimental import pallas as pl
from jax.experimental.pallas import tpu as pltpu

ABLATE = int(os.environ.get("ABLATE", "0"))
SCOPES = os.environ.get("SCOPES") == "1"

import contextlib


def _scope(name):
    return jax.named_scope(name) if SCOPES else contextlib.nullcontext()
COMM_ON = ABLATE in (0, 5, 6, 7, 8, 9)
LIVE_SEND = {6: [1, 3], 7: [1], 9: [1]}.get(ABLATE, [2, 1, 3])
LIVE_RECV = {6: [3, 1], 7: [3], 9: [3]}.get(ABLATE, [2, 3, 1])
WHOLE = ABLATE in (8, 9)

N_DEV = 4
XCH = 4

_GELU_C = 0.7978845608028654


def _gelu(y):
    return 0.5 * y * (1.0 + jnp.tanh(_GELU_C * (y + 0.044715 * y * y * y)))


def kernel(x, w_mat):
    m_per, k = x.shape
    _, n = w_mat.shape
    n_per = n // N_DEV
    xrows = m_per // XCH

    def body(
        x_hbm,
        w_hbm,
        out_ref,
        x_bf,
        w_bf,
        xs,
        ws,
        x_sems,
        w_sems,
        send_buf,
        recv_buf,
        send_sems,
        recv_sems,
        out_stage,
        out_sems,
    ):
        me = lax.axis_index("i")

        def xdma(ci, slot):
            return pltpu.make_async_copy(
                x_hbm.at[pl.ds(ci * xrows, xrows), :],
                xs.at[slot],
                x_sems.at[slot],
            )

        def odma(row0, rows, o, r):
            return pltpu.make_async_copy(
                out_stage.at[pl.ds(row0, rows), :],
                out_ref.at[pl.ds(row0, rows), :],
                out_sems.at[o, r],
            )

        def wdma(c, slot):
            return pltpu.make_async_copy(
                w_hbm.at[:, pl.ds(c * n_per, n_per)],
                ws.at[slot],
                w_sems.at[slot, 0],
            )

        half = n_per // 2

        def wdma_h(c, h, slot=0):
            return pltpu.make_async_copy(
                w_hbm.at[:, pl.ds(c * n_per + h * half, half)],
                ws.at[slot, :, pl.ds(h * half, half)],
                w_sems.at[slot, h],
            )

        if ABLATE < 5:
            wdma_h((me + 2) % N_DEV, 0).start()
            xdma(0, 0).start()
            wdma_h((me + 2) % N_DEV, 1).start()
            xdma(1, 1).start()

        barrier_sem = pltpu.get_barrier_semaphore()
        for d in range(1, N_DEV):
            pl.semaphore_signal(
                barrier_sem,
                inc=1,
                device_id=((me + d) % N_DEV,),
                device_id_type=pl.DeviceIdType.MESH,
            )
        pl.semaphore_wait(barrier_sem, N_DEV - 1)

        offs = [2, 1, 3, 0]
        if ABLATE >= 5:
            for idx, off in enumerate(LIVE_SEND):
                c = (me + off) % N_DEV
                if WHOLE:
                    pltpu.make_async_remote_copy(
                        src_ref=send_buf.at[idx],
                        dst_ref=recv_buf.at[me],
                        send_sem=send_sems.at[idx, 0],
                        recv_sem=recv_sems.at[me, 0],
                        device_id=(c,),
                        device_id_type=pl.DeviceIdType.MESH,
                    ).start()
                else:
                    for r in range(XCH):
                        pltpu.make_async_remote_copy(
                            src_ref=send_buf.at[idx, r],
                            dst_ref=recv_buf.at[me, r],
                            send_sem=send_sems.at[idx, r],
                            recv_sem=recv_sems.at[me, r],
                            device_id=(c,),
                            device_id_type=pl.DeviceIdType.MESH,
                        ).start()
        for idx, off in enumerate(offs if ABLATE < 5 else []):
            c = (me + off) % N_DEV
            slot = idx % 2
            if idx + 1 < N_DEV:
                nxt = (me + offs[idx + 1]) % N_DEV
                if offs[idx + 1] == 0:
                    wdma_h(nxt, 0, 1 - slot).start()
                    wdma_h(nxt, 1, 1 - slot).start()
                else:
                    wdma((me + offs[idx + 1]) % N_DEV, 1 - slot).start()
            if off == 0:
                for h in range(2):
                    cols = slice(h * half, (h + 1) * half)
                    with _scope(f"wwait#own_h{h}"):
                        wdma_h(c, h, slot).wait()
                    with _scope(f"wconv#own_h{h}"):
                        w_bf[slot, :, cols] = ws[slot][:, cols].astype(
                            jnp.bfloat16
                        )
                    with _scope(f"dot#own_h{h}"):
                        yh = _gelu(
                            jnp.dot(
                                x_bf[...],
                                w_bf[slot][:, cols],
                                preferred_element_type=jnp.float32,
                            )
                        )
                    with _scope(f"ownstore#h{h}"):
                        out_stage[pl.ds(me * m_per, m_per), cols] = yh
                        pltpu.make_async_copy(
                            out_stage.at[pl.ds(me * m_per, m_per), cols],
                            out_ref.at[pl.ds(me * m_per, m_per), cols],
                            out_sems.at[3, h],
                        ).start()
                continue
            if idx == 0:
                with _scope("wwait#idx=0a"):
                    wdma_h(c, 0).wait()
                with _scope("wconv#idx=0a"):
                    w_bf[0, :, :half] = ws[0][:, :half].astype(jnp.bfloat16)
            else:
                with _scope(f"wwait#idx={idx}"):
                    wdma(c, slot).wait()
                with _scope(f"wconv#idx={idx}"):
                    w_bf[slot] = ws[slot].astype(jnp.bfloat16)

            for r in range(XCH):
                if idx == 0:
                    xslot = r % 2
                    with _scope(f"xwait#r={r}"):
                        xdma(r, xslot).wait()
                    with _scope(f"xconv#r={r}"):
                        x_bf[pl.ds(r * xrows, xrows), :] = xs[xslot].astype(
                            jnp.bfloat16
                        )
                    if r + 2 < XCH:
                        xdma(r + 2, xslot).start()

                if idx == 0 and r == 0:
                    with _scope("dot#idx=0_r=0a"):
                        ya = _gelu(
                            jnp.dot(
                                x_bf[pl.ds(0, xrows), :],
                                w_bf[0, :, :half],
                                preferred_element_type=jnp.float32,
                            )
                        )
                    with _scope("wwait#idx=0b"):
                        wdma_h(c, 1).wait()
                    with _scope("wconv#idx=0b"):
                        w_bf[0, :, half:] = ws[0][:, half:].astype(
                            jnp.bfloat16
                        )
                    with _scope("dot#idx=0_r=0b"):
                        yb = _gelu(
                            jnp.dot(
                                x_bf[pl.ds(0, xrows), :],
                                w_bf[0, :, half:],
                                preferred_element_type=jnp.float32,
                            )
                        )
                    y = jnp.concatenate([ya, yb], axis=1)
                else:
                    with _scope(f"dot#idx={idx}_r={r}"):
                        y = jnp.dot(
                            x_bf[pl.ds(r * xrows, xrows), :],
                            w_bf[slot],
                            preferred_element_type=jnp.float32,
                        )
                        y = _gelu(y)

                if True:
                    with _scope(f"sendcast#idx={idx}_r={r}"):
                        send_buf[idx, r] = y.astype(jnp.bfloat16)
                    if COMM_ON:
                        pltpu.make_async_remote_copy(
                            src_ref=send_buf.at[idx, r],
                            dst_ref=recv_buf.at[me, r],
                            send_sem=send_sems.at[idx, r],
                            recv_sem=recv_sems.at[me, r],
                            device_id=(c,),
                            device_id_type=pl.DeviceIdType.MESH,
                        ).start()

        for off in LIVE_RECV:
            s = (me + off) % N_DEV
            sub = ABLATE < 8
            if COMM_ON and not sub:
                with _scope(f"recvwait#off={off}"):
                    pltpu.make_async_remote_copy(
                        src_ref=send_buf.at[0],
                        dst_ref=recv_buf.at[s],
                        send_sem=send_sems.at[0, 0],
                        recv_sem=recv_sems.at[s, 0],
                        device_id=(s,),
                        device_id_type=pl.DeviceIdType.MESH,
                    ).wait_recv()
            for r in range(XCH):
                if COMM_ON and sub:
                    with _scope(f"recvwait#off={off}_r={r}"):
                        pltpu.make_async_remote_copy(
                            src_ref=send_buf.at[0, r],
                            dst_ref=recv_buf.at[s, r],
                            send_sem=send_sems.at[0, r],
                            recv_sem=recv_sems.at[s, r],
                            device_id=(s,),
                            device_id_type=pl.DeviceIdType.MESH,
                        ).wait_recv()
                with _scope(f"recvstore#off={off}_r={r}"):
                    out_stage[
                        pl.ds(s * m_per + r * xrows, xrows), :
                    ] = recv_buf[s, r].astype(jnp.float32)
                    odma(s * m_per + r * xrows, xrows, off % 3, r).start()

        if COMM_ON:
            for idx, off in enumerate(LIVE_SEND):
                c = (me + off) % N_DEV
                if WHOLE:
                    pltpu.make_async_remote_copy(
                        src_ref=send_buf.at[idx],
                        dst_ref=recv_buf.at[me],
                        send_sem=send_sems.at[idx, 0],
                        recv_sem=recv_sems.at[me, 0],
                        device_id=(c,),
                        device_id_type=pl.DeviceIdType.MESH,
                    ).wait_send()
                    continue
                for r in range(XCH):
                    pltpu.make_async_remote_copy(
                        src_ref=send_buf.at[idx, r],
                        dst_ref=recv_buf.at[me, r],
                        send_sem=send_sems.at[idx, r],
                        recv_sem=recv_sems.at[me, r],
                        device_id=(c,),
                        device_id_type=pl.DeviceIdType.MESH,
                    ).wait_send()

        for off in LIVE_RECV:
            s_pos = (me + off) % N_DEV
            for r in range(XCH):
                odma(s_pos * m_per + r * xrows, xrows, off % 3, r).wait()
        if ABLATE < 5:
            for h in range(2):
                pltpu.make_async_copy(
                    out_stage.at[pl.ds(me * m_per, m_per), pl.ds(h * half, half)],
                    out_ref.at[pl.ds(me * m_per, m_per), pl.ds(h * half, half)],
                    out_sems.at[3, h],
                ).wait()

    return pl.pallas_call(
        body,
        out_shape=jax.ShapeDtypeStruct((N_DEV * m_per, n_per), jnp.float32),
        in_specs=[
            pl.BlockSpec(memory_space=pltpu.MemorySpace.HBM),
            pl.BlockSpec(memory_space=pltpu.MemorySpace.HBM),
        ],
        out_specs=pl.BlockSpec(memory_space=pltpu.MemorySpace.HBM),
        scratch_shapes=[
            pltpu.VMEM((m_per, k), jnp.bfloat16),
            pltpu.VMEM((2, k, n_per), jnp.bfloat16),
            pltpu.VMEM((2, xrows, k), jnp.float32),
            pltpu.VMEM((2, k, n_per), jnp.float32),
            pltpu.SemaphoreType.DMA((2,)),
            pltpu.SemaphoreType.DMA((2, 2)),
            pltpu.VMEM((N_DEV, XCH, xrows, n_per), jnp.bfloat16),
            pltpu.VMEM((N_DEV, XCH, xrows, n_per), jnp.bfloat16),
            pltpu.SemaphoreType.DMA((N_DEV, XCH)),
            pltpu.SemaphoreType.DMA((N_DEV, XCH)),
            pltpu.VMEM((N_DEV * m_per, n_per), jnp.float32),
            pltpu.SemaphoreType.DMA((N_DEV, XCH)),
        ],
        compiler_params=pltpu.CompilerParams(
            collective_id=0,
            vmem_limit_bytes=100 * 1024 * 1024,
        ),
    )(x, w_mat)
